# baseline (speedup 1.0000x reference)
import functools

import jax
import jax.numpy as jnp
from jax import lax
from jax.experimental import pallas as pl
from jax.experimental.pallas import tpu as pltpu
from jax.experimental.pallas import tpu_sc as plsc

SEQ = 200
D = 64
BATCH = 4096
NW = 32
SEQ_PER_W = BATCH // NW
SEQ_PER_CHUNK = 8
TOK_PER_CHUNK = SEQ_PER_CHUNK * SEQ
CHUNKS = SEQ_PER_W // SEQ_PER_CHUNK
TOK_PER_W = SEQ_PER_W * SEQ
GSUB = 128


def _sc_embed(ids_flat, emb, pos):
    mesh = plsc.VectorSubcoreMesh(core_axis_name="c", subcore_axis_name="s")

    @functools.partial(
        pl.kernel,
        mesh=mesh,
        out_type=jax.ShapeDtypeStruct((BATCH * SEQ, D), jnp.float32),
        scratch_types=[
            pltpu.VMEM((TOK_PER_CHUNK,), jnp.int32),
            pltpu.VMEM((TOK_PER_CHUNK, D), jnp.float32),
            pltpu.VMEM((SEQ, D), jnp.float32),
            pltpu.SemaphoreType.DMA,
        ],
        compiler_params=pltpu.CompilerParams(use_tc_tiling_on_sc=False),
    )
    def k(ids_hbm, emb_hbm, pos_hbm, out_hbm, idx_v, rows_v, pos_v, sem):
        wid = lax.axis_index("s") * 2 + lax.axis_index("c")
        base = wid * TOK_PER_W

        pltpu.sync_copy(pos_hbm.at[pl.ds(0, SEQ)], pos_v)

        def chunk_body(c, carry):
            tok0 = base + c * TOK_PER_CHUNK
            pltpu.sync_copy(ids_hbm.at[pl.ds(tok0, TOK_PER_CHUNK)], idx_v)

            handles = []
            off = 0
            while off < TOK_PER_CHUNK:
                n = min(GSUB, TOK_PER_CHUNK - off)
                handles.append(pltpu.async_copy(
                    emb_hbm.at[idx_v.at[pl.ds(off, n)]],
                    rows_v.at[pl.ds(off, n)],
                    sem,
                ))
                off += n
            for h in handles:
                h.wait()

            def pos_body(p, carry2):
                for j in range(D // 16):
                    pv = pos_v[p, pl.ds(16 * j, 16)]
                    for s in range(SEQ_PER_CHUNK):
                        rows_v[s * SEQ + p, pl.ds(16 * j, 16)] += pv
                return carry2

            lax.fori_loop(0, SEQ, pos_body, 0)

            pltpu.sync_copy(rows_v, out_hbm.at[pl.ds(tok0, TOK_PER_CHUNK)])
            return carry

        lax.fori_loop(0, CHUNKS, chunk_body, 0)

    return k(ids_flat, emb, pos)


def kernel(input_ids, embedding, position_embedding):
    ids_flat = input_ids.reshape(-1).astype(jnp.int32)
    out = _sc_embed(ids_flat, embedding, position_embedding)
    return out.reshape(BATCH, SEQ, D)
